# baseline (speedup 1.0000x reference)
import functools

import jax
import jax.numpy as jnp
from jax import lax
from jax.experimental import pallas as pl
from jax.experimental.pallas import tpu as pltpu
from jax.experimental.pallas import tpu_sc as plsc

NC = 2
NS = 16
NW = NC * NS

B = 4096
L = 200
D = 64
TOTAL = B * L
PER_W = TOTAL // NW
G2 = 512
NCH = PER_W // G2


def _gather_kernel(table_hbm, idx_hbm, out_hbm, idx_v, rows_v, gsem):
    c = lax.axis_index("c")
    s = lax.axis_index("s")
    wid = s * NC + c
    pltpu.sync_copy(idx_hbm.at[wid], idx_v)
    base = wid * PER_W

    def body(j, carry):
        pltpu.async_copy(table_hbm.at[idx_v.at[pl.ds(j * G2, G2)]],
                         rows_v, gsem).wait()
        pltpu.sync_copy(rows_v, out_hbm.at[pl.ds(base + j * G2, G2)])
        return carry

    lax.fori_loop(0, NCH, body, 0)


@jax.jit
def _run(table, idx2):
    k = functools.partial(
        pl.kernel,
        out_type=jax.ShapeDtypeStruct((TOTAL, D), jnp.float32),
        mesh=plsc.VectorSubcoreMesh(core_axis_name="c", subcore_axis_name="s"),
        scratch_types=[
            pltpu.VMEM((PER_W,), jnp.int32),
            pltpu.VMEM((G2, D), jnp.float32),
            pltpu.SemaphoreType.DMA,
        ],
        compiler_params=pltpu.CompilerParams(use_tc_tiling_on_sc=False),
    )(_gather_kernel)
    return k(table, idx2)


def kernel(batch, table):
    idx2 = batch.reshape(NW, PER_W)
    out = _run(table, idx2)
    return out.reshape(B, L, D)


# speedup vs baseline: 1.0269x; 1.0269x over previous
import functools

import jax
import jax.numpy as jnp
from jax import lax
from jax.experimental import pallas as pl
from jax.experimental.pallas import tpu as pltpu
from jax.experimental.pallas import tpu_sc as plsc

NC = 2
NS = 16
NW = NC * NS

B = 4096
L = 200
D = 64
TOTAL = B * L
PER_W = TOTAL // NW
G2 = 512
NCH = PER_W // G2
R = 3
FD = 2


def _gather_kernel(table_hbm, idx_hbm, out_hbm, idx_v, rows_v, gsem, osem):
    c = lax.axis_index("c")
    s = lax.axis_index("s")
    wid = s * NC + c
    pltpu.sync_copy(idx_hbm.at[wid], idx_v)
    base = wid * PER_W

    def fire_gather(j, slot):
        pltpu.async_copy(table_hbm.at[idx_v.at[pl.ds(j * G2, G2)]],
                         rows_v.at[slot], gsem.at[slot])

    def wait_gather(slot):
        pltpu.make_async_copy(table_hbm.at[idx_v.at[pl.ds(0, G2)]],
                              rows_v.at[slot], gsem.at[slot]).wait()

    def fire_out(j, slot):
        pltpu.async_copy(rows_v.at[slot],
                         out_hbm.at[pl.ds(base + j * G2, G2)], osem.at[slot])

    def wait_out(slot):
        pltpu.make_async_copy(rows_v.at[slot],
                              out_hbm.at[pl.ds(base, G2)], osem.at[slot]).wait()

    for p in range(FD):
        fire_gather(p, p)

    def body(j, carry):
        slot = lax.rem(j, R)
        wait_gather(slot)
        fire_out(j, slot)

        @pl.when(j < NCH - FD)
        def _fire_next():
            f = j + FD
            slot2 = lax.rem(f, R)

            @pl.when(f >= R)
            def _recycle():
                wait_out(slot2)

            fire_gather(f, slot2)

        return carry

    lax.fori_loop(0, NCH, body, 0)

    for p in range(R):
        wait_out((NCH - R + p) % R)


@jax.jit
def _run(table, idx2):
    k = functools.partial(
        pl.kernel,
        out_type=jax.ShapeDtypeStruct((TOTAL, D), jnp.float32),
        mesh=plsc.VectorSubcoreMesh(core_axis_name="c", subcore_axis_name="s"),
        scratch_types=[
            pltpu.VMEM((PER_W,), jnp.int32),
            pltpu.VMEM((R, G2, D), jnp.float32),
            pltpu.SemaphoreType.DMA((R,)),
            pltpu.SemaphoreType.DMA((R,)),
        ],
        compiler_params=pltpu.CompilerParams(use_tc_tiling_on_sc=False),
    )(_gather_kernel)
    return k(table, idx2)


def kernel(batch, table):
    idx2 = batch.reshape(NW, PER_W)
    out = _run(table, idx2)
    return out.reshape(B, L, D)


# speedup vs baseline: 1.2523x; 1.2195x over previous
import functools

import jax
import jax.numpy as jnp
from jax import lax
from jax.experimental import pallas as pl
from jax.experimental.pallas import tpu as pltpu
from jax.experimental.pallas import tpu_sc as plsc

NC = 2
NS = 16
NW = NC * NS

B = 4096
L = 200
D = 64
DP = 128
TOTAL = B * L
PER_W = TOTAL // NW
G2 = 256
NCH = PER_W // G2
R = 3
FD = 2


def _gather_kernel(table_hbm, idx_hbm, out_hbm, idx_v, rows_v, gsem, osem):
    c = lax.axis_index("c")
    s = lax.axis_index("s")
    wid = s * NC + c
    pltpu.sync_copy(idx_hbm.at[wid], idx_v)
    base = wid * PER_W

    def fire_gather(j, slot):
        pltpu.async_copy(table_hbm.at[idx_v.at[pl.ds(j * G2, G2)]],
                         rows_v.at[slot], gsem.at[slot])

    def wait_gather(slot):
        pltpu.make_async_copy(table_hbm.at[idx_v.at[pl.ds(0, G2)]],
                              rows_v.at[slot], gsem.at[slot]).wait()

    def fire_out(j, slot):
        pltpu.async_copy(rows_v.at[slot],
                         out_hbm.at[pl.ds(base + j * G2, G2)], osem.at[slot])

    def wait_out(slot):
        pltpu.make_async_copy(rows_v.at[slot],
                              out_hbm.at[pl.ds(base, G2)], osem.at[slot]).wait()

    for p in range(FD):
        fire_gather(p, p)

    def body(j, carry):
        slot = lax.rem(j, R)
        wait_gather(slot)
        fire_out(j, slot)

        @pl.when(j < NCH - FD)
        def _fire_next():
            f = j + FD
            slot2 = lax.rem(f, R)

            @pl.when(f >= R)
            def _recycle():
                wait_out(slot2)

            fire_gather(f, slot2)

        return carry

    lax.fori_loop(0, NCH, body, 0)

    for p in range(R):
        wait_out((NCH - R + p) % R)


@jax.jit
def _run(table, idx2):
    k = functools.partial(
        pl.kernel,
        out_type=jax.ShapeDtypeStruct((TOTAL, DP), jnp.float32),
        mesh=plsc.VectorSubcoreMesh(core_axis_name="c", subcore_axis_name="s"),
        scratch_types=[
            pltpu.VMEM((PER_W,), jnp.int32),
            pltpu.VMEM((R, G2, DP), jnp.float32),
            pltpu.SemaphoreType.DMA((R,)),
            pltpu.SemaphoreType.DMA((R,)),
        ],
    )(_gather_kernel)
    return k(table, idx2)


def kernel(batch, table):
    idx2 = batch.reshape(NW, PER_W)
    table_p = jnp.pad(table, ((0, 0), (0, DP - D)))
    out = _run(table_p, idx2)
    return out[:, :D].reshape(B, L, D)
